# baseline (speedup 1.0000x reference)
import functools

import jax
import jax.numpy as jnp
from jax.experimental import pallas as pl
from jax.experimental.pallas import tpu as pltpu

_EPS = 0.1
_ROW_BLK = 512
_VOCAB_BLK = 4096


def _loss_body(nv, num_classes, num_rows, t_ref, x_ref, out_ref,
               m_ref, s_ref, su_ref, pt_ref):
    r = pl.program_id(0)
    j = pl.program_id(1)
    rb = x_ref.shape[0]
    vb = x_ref.shape[1]

    @pl.when(j == 0)
    def _init_acc():
        m_ref[...] = jnp.full((rb, 1), -jnp.inf, jnp.float32)
        s_ref[...] = jnp.zeros((rb, 1), jnp.float32)
        su_ref[...] = jnp.zeros((rb, 1), jnp.float32)
        pt_ref[...] = jnp.zeros((rb, 1), jnp.float32)

    t = t_ref[...]
    cols = j * vb + jax.lax.broadcasted_iota(jnp.int32, (1, vb), 1)
    tmask = cols == t

    ones = jnp.ones((vb, 1), jnp.float32)
    dot_dims = (((1,), (0,)), ((), ()))

    def rowsum(y):
        return jax.lax.dot_general(y, ones, dot_dims,
                                   preferred_element_type=jnp.float32)

    def update(x, xm):
        bm = jnp.max(xm, axis=1, keepdims=True)
        m_new = jnp.maximum(m_ref[...], bm)
        alpha = jnp.exp(m_ref[...] - m_new)
        s_ref[...] = s_ref[...] * alpha + rowsum(jnp.exp(xm - m_new))
        su_ref[...] += rowsum(x)
        pt_ref[...] += rowsum(jnp.where(tmask, x, 0.0))
        m_ref[...] = m_new

    @pl.when(j < nv - 1)
    def _full_block():
        x = x_ref[...]
        update(x, x)

    @pl.when(j == nv - 1)
    def _last_block():
        x = x_ref[...]
        valid = cols < num_classes
        xm = jnp.where(valid, x, -jnp.inf)
        x0 = jnp.where(valid, x, 0.0)
        update(x0, xm)
        a = _EPS / (num_classes - 1)
        c = 1.0 - _EPS - a
        lse = m_ref[...] + jnp.log(s_ref[...])
        loss_rows = (a * num_classes + c) * lse - (
            a * su_ref[...] + c * pt_ref[...])
        out_ref[...] = jnp.sum(loss_rows).reshape(1, 1, 1)


def kernel(pred, target):
    n, k = pred.shape
    rb = _ROW_BLK
    vb = _VOCAB_BLK
    nr = n // rb
    nv = -(-k // vb)

    t2 = target.astype(jnp.int32).reshape(n, 1)
    body = functools.partial(_loss_body, nv, k, n)
    out = pl.pallas_call(
        body,
        grid=(nr, nv),
        in_specs=[
            pl.BlockSpec((rb, 1), lambda r, j: (r, 0)),
            pl.BlockSpec((rb, vb), lambda r, j: (r, j)),
        ],
        out_specs=pl.BlockSpec((1, 1, 1), lambda r, j: (r, 0, 0)),
        out_shape=jax.ShapeDtypeStruct((nr, 1, 1), jnp.float32),
        scratch_shapes=[
            pltpu.VMEM((rb, 1), jnp.float32),
            pltpu.VMEM((rb, 1), jnp.float32),
            pltpu.VMEM((rb, 1), jnp.float32),
            pltpu.VMEM((rb, 1), jnp.float32),
        ],
        compiler_params=pltpu.CompilerParams(
            dimension_semantics=("parallel", "arbitrary")),
    )(t2, pred)
    return jnp.sum(out) / n


# speedup vs baseline: 1.0825x; 1.0825x over previous
import functools

import jax
import jax.numpy as jnp
from jax.experimental import pallas as pl
from jax.experimental.pallas import tpu as pltpu

_EPS = 0.1
_ROW_BLK = 1024
_VOCAB_BLK = 2048


def _loss_body(nv, num_classes, num_rows, t_ref, x_ref, out_ref,
               m_ref, s_ref, su_ref, pt_ref):
    r = pl.program_id(0)
    j = pl.program_id(1)
    rb = x_ref.shape[0]
    vb = x_ref.shape[1]

    @pl.when(j == 0)
    def _init_acc():
        m_ref[...] = jnp.full((rb, 1), -jnp.inf, jnp.float32)
        s_ref[...] = jnp.zeros((rb, 1), jnp.float32)
        su_ref[...] = jnp.zeros((rb, 1), jnp.float32)
        pt_ref[...] = jnp.zeros((rb, 1), jnp.float32)

    t = t_ref[...]
    cols = j * vb + jax.lax.broadcasted_iota(jnp.int32, (1, vb), 1)
    tmask = cols == t

    def update(x, xm):
        bm = jnp.max(xm, axis=1, keepdims=True)
        m_new = jnp.maximum(m_ref[...], bm)
        alpha = jnp.exp(m_ref[...] - m_new)
        s_ref[...] = s_ref[...] * alpha + jnp.sum(
            jnp.exp(xm - m_new), axis=1, keepdims=True)
        su_ref[...] += jnp.sum(x, axis=1, keepdims=True)
        pt_ref[...] += jnp.sum(jnp.where(tmask, x, 0.0), axis=1,
                               keepdims=True)
        m_ref[...] = m_new

    @pl.when(j < nv - 1)
    def _full_block():
        x = x_ref[...]
        update(x, x)

    @pl.when(j == nv - 1)
    def _last_block():
        x = x_ref[...]
        valid = cols < num_classes
        xm = jnp.where(valid, x, -jnp.inf)
        x0 = jnp.where(valid, x, 0.0)
        update(x0, xm)
        a = _EPS / (num_classes - 1)
        c = 1.0 - _EPS - a
        lse = m_ref[...] + jnp.log(s_ref[...])
        loss_rows = (a * num_classes + c) * lse - (
            a * su_ref[...] + c * pt_ref[...])
        out_ref[...] = jnp.sum(loss_rows).reshape(1, 1, 1)


def kernel(pred, target):
    n, k = pred.shape
    rb = _ROW_BLK
    vb = _VOCAB_BLK
    nr = n // rb
    nv = -(-k // vb)

    t2 = target.astype(jnp.int32).reshape(n, 1)
    body = functools.partial(_loss_body, nv, k, n)
    out = pl.pallas_call(
        body,
        grid=(nr, nv),
        in_specs=[
            pl.BlockSpec((rb, 1), lambda r, j: (r, 0)),
            pl.BlockSpec((rb, vb), lambda r, j: (r, j)),
        ],
        out_specs=pl.BlockSpec((1, 1, 1), lambda r, j: (r, 0, 0)),
        out_shape=jax.ShapeDtypeStruct((nr, 1, 1), jnp.float32),
        scratch_shapes=[
            pltpu.VMEM((rb, 1), jnp.float32),
            pltpu.VMEM((rb, 1), jnp.float32),
            pltpu.VMEM((rb, 1), jnp.float32),
            pltpu.VMEM((rb, 1), jnp.float32),
        ],
        compiler_params=pltpu.CompilerParams(
            dimension_semantics=("parallel", "arbitrary")),
    )(t2, pred)
    return jnp.sum(out) / n
